# baseline (device time: 463620 ns/iter reference)
import jax
import jax.numpy as jnp
from jax import lax
from jax.experimental import pallas as pl
from jax.experimental.pallas import tpu as pltpu

N_DEV = 16


def kernel(x, w_mat):
    m, k_per = x.shape
    _, n = w_mat.shape
    m_per = m // N_DEV
    n_half = n // 2

    def body(x_ref, w_ref, out_ref,
             send_cw, recv_cw, send_ccw, recv_ccw,
             send_sems, recv_sems, credit_cw, credit_ccw,
             amax_buf, amax_send_sems, amax_recv_sems, exit_sem):
        me = lax.axis_index("i")
        left = lax.rem(me + N_DEV - 1, N_DEV)
        right = lax.rem(me + 1, N_DEV)

        barrier = pltpu.get_barrier_semaphore()
        for nbr in (left, right):
            pl.semaphore_signal(barrier, inc=1, device_id=(nbr,),
                                device_id_type=pl.DeviceIdType.MESH)
        pl.semaphore_wait(barrier, 2)

        def partial(c, lo):
            xa = x_ref[pl.ds(c * m_per, m_per), :]
            return jnp.dot(xa, w_ref[:, lo:lo + n_half],
                           preferred_element_type=jnp.float32)

        for h in range(N_DEV - 1):
            c_cw = lax.rem(me + 2 * N_DEV - 1 - h, N_DEV)
            c_ccw = lax.rem(me + 1 + h, N_DEV)
            p_cw = partial(c_cw, 0)
            p_ccw = partial(c_ccw, n_half)
            if h == 0:
                send_cw[...] = p_cw.astype(jnp.bfloat16)
                send_ccw[...] = p_ccw.astype(jnp.bfloat16)
            else:
                send_cw[...] = (p_cw + recv_cw[...].astype(jnp.float32)
                                ).astype(jnp.bfloat16)
                send_ccw[...] = (p_ccw + recv_ccw[...].astype(jnp.float32)
                                 ).astype(jnp.bfloat16)
                pl.semaphore_signal(credit_cw, inc=1, device_id=(left,),
                                    device_id_type=pl.DeviceIdType.MESH)
                pl.semaphore_signal(credit_ccw, inc=1, device_id=(right,),
                                    device_id_type=pl.DeviceIdType.MESH)
                pl.semaphore_wait(credit_cw, 1)
                pl.semaphore_wait(credit_ccw, 1)
            rdma_cw = pltpu.make_async_remote_copy(
                src_ref=send_cw, dst_ref=recv_cw,
                send_sem=send_sems.at[0], recv_sem=recv_sems.at[0],
                device_id=(right,), device_id_type=pl.DeviceIdType.MESH)
            rdma_ccw = pltpu.make_async_remote_copy(
                src_ref=send_ccw, dst_ref=recv_ccw,
                send_sem=send_sems.at[1], recv_sem=recv_sems.at[1],
                device_id=(left,), device_id_type=pl.DeviceIdType.MESH)
            rdma_cw.start()
            rdma_ccw.start()
            rdma_cw.wait()
            rdma_ccw.wait()

        out_ref[:, :n_half] = partial(me, 0) + recv_cw[...].astype(jnp.float32)
        out_ref[:, n_half:] = (partial(me, n_half)
                               + recv_ccw[...].astype(jnp.float32))

        local_amax = jnp.max(jnp.abs(out_ref[...]))
        amax_buf[me] = jnp.full((8, 128), local_amax, jnp.float32)
        sends = []
        for o in range(1, N_DEV):
            tgt = lax.rem(me + o, N_DEV)
            s = pltpu.make_async_remote_copy(
                src_ref=amax_buf.at[me], dst_ref=amax_buf.at[me],
                send_sem=amax_send_sems.at[o], recv_sem=amax_recv_sems.at[me],
                device_id=(tgt,), device_id_type=pl.DeviceIdType.MESH)
            s.start()
            sends.append(s)
        for o in range(1, N_DEV):
            src_dev = lax.rem(me + N_DEV - o, N_DEV)
            r = pltpu.make_async_remote_copy(
                src_ref=amax_buf.at[me], dst_ref=amax_buf.at[src_dev],
                send_sem=amax_send_sems.at[0],
                recv_sem=amax_recv_sems.at[src_dev],
                device_id=(left,), device_id_type=pl.DeviceIdType.MESH)
            r.wait_recv()

        g_amax = jnp.max(amax_buf[...])
        scale = g_amax / 127.0
        q = jnp.clip(jnp.round(out_ref[...] / scale), -127.0, 127.0)
        out_ref[...] = q * scale

        for s in sends:
            s.wait_send()

        for nbr in (left, right):
            pl.semaphore_signal(exit_sem, inc=1, device_id=(nbr,),
                                device_id_type=pl.DeviceIdType.MESH)
        pl.semaphore_wait(exit_sem, 2)

    return pl.pallas_call(
        body,
        out_shape=jax.ShapeDtypeStruct((m_per, n), jnp.float32),
        in_specs=[pl.BlockSpec(memory_space=pltpu.VMEM),
                  pl.BlockSpec(memory_space=pltpu.VMEM)],
        out_specs=pl.BlockSpec(memory_space=pltpu.VMEM),
        scratch_shapes=[
            pltpu.VMEM((m_per, n_half), jnp.bfloat16),
            pltpu.VMEM((m_per, n_half), jnp.bfloat16),
            pltpu.VMEM((m_per, n_half), jnp.bfloat16),
            pltpu.VMEM((m_per, n_half), jnp.bfloat16),
            pltpu.SemaphoreType.DMA((2,)),
            pltpu.SemaphoreType.DMA((2,)),
            pltpu.SemaphoreType.REGULAR,
            pltpu.SemaphoreType.REGULAR,
            pltpu.VMEM((N_DEV, 8, 128), jnp.float32),
            pltpu.SemaphoreType.DMA((N_DEV,)),
            pltpu.SemaphoreType.DMA((N_DEV,)),
            pltpu.SemaphoreType.REGULAR,
        ],
        compiler_params=pltpu.CompilerParams(
            collective_id=0,
            vmem_limit_bytes=100 * 1024 * 1024,
        ),
    )(x, w_mat)


# device time: 463461 ns/iter; 1.0003x vs baseline; 1.0003x over previous
import jax
import jax.numpy as jnp
from jax import lax
from jax.experimental import pallas as pl
from jax.experimental.pallas import tpu as pltpu

N_DEV = 16


def kernel(x, w_mat):
    m, k_per = x.shape
    _, n = w_mat.shape
    m_per = m // N_DEV
    n_half = n // 2

    def body(x_ref, w_ref, out_ref,
             send_cw, recv_cw, send_ccw, recv_ccw,
             send_sems, recv_sems, credit_cw, credit_ccw,
             amax_buf, amax_send_sems, amax_recv_sems, exit_sem):
        me = lax.axis_index("i")
        left = lax.rem(me + N_DEV - 1, N_DEV)
        right = lax.rem(me + 1, N_DEV)

        barrier = pltpu.get_barrier_semaphore()
        for nbr in (left, right):
            pl.semaphore_signal(barrier, inc=1, device_id=(nbr,),
                                device_id_type=pl.DeviceIdType.MESH)
        pl.semaphore_wait(barrier, 2)

        def partial(c, lo):
            xa = x_ref[pl.ds(c * m_per, m_per), :]
            return jnp.dot(xa, w_ref[:, lo:lo + n_half],
                           preferred_element_type=jnp.float32)

        def make_rdmas():
            rdma_cw = pltpu.make_async_remote_copy(
                src_ref=send_cw, dst_ref=recv_cw,
                send_sem=send_sems.at[0], recv_sem=recv_sems.at[0],
                device_id=(right,), device_id_type=pl.DeviceIdType.MESH)
            rdma_ccw = pltpu.make_async_remote_copy(
                src_ref=send_ccw, dst_ref=recv_ccw,
                send_sem=send_sems.at[1], recv_sem=recv_sems.at[1],
                device_id=(left,), device_id_type=pl.DeviceIdType.MESH)
            return rdma_cw, rdma_ccw

        send_cw[...] = partial(lax.rem(me + N_DEV - 1, N_DEV),
                               0).astype(jnp.bfloat16)
        send_ccw[...] = partial(lax.rem(me + 1, N_DEV),
                                n_half).astype(jnp.bfloat16)
        prev_cw, prev_ccw = make_rdmas()
        prev_cw.start()
        prev_ccw.start()

        for h in range(1, N_DEV - 1):
            c_cw = lax.rem(me + 2 * N_DEV - 1 - h, N_DEV)
            c_ccw = lax.rem(me + 1 + h, N_DEV)
            p_cw = partial(c_cw, 0)
            p_ccw = partial(c_ccw, n_half)
            prev_cw.wait()
            prev_ccw.wait()
            send_cw[...] = (p_cw + recv_cw[...].astype(jnp.float32)
                            ).astype(jnp.bfloat16)
            send_ccw[...] = (p_ccw + recv_ccw[...].astype(jnp.float32)
                             ).astype(jnp.bfloat16)
            pl.semaphore_signal(credit_cw, inc=1, device_id=(left,),
                                device_id_type=pl.DeviceIdType.MESH)
            pl.semaphore_signal(credit_ccw, inc=1, device_id=(right,),
                                device_id_type=pl.DeviceIdType.MESH)
            pl.semaphore_wait(credit_cw, 1)
            pl.semaphore_wait(credit_ccw, 1)
            prev_cw, prev_ccw = make_rdmas()
            prev_cw.start()
            prev_ccw.start()

        p_me_cw = partial(me, 0)
        p_me_ccw = partial(me, n_half)
        prev_cw.wait()
        prev_ccw.wait()
        y_cw = p_me_cw + recv_cw[...].astype(jnp.float32)
        y_ccw = p_me_ccw + recv_ccw[...].astype(jnp.float32)
        out_ref[:, :n_half] = y_cw
        out_ref[:, n_half:] = y_ccw

        local_amax = jnp.maximum(jnp.max(jnp.abs(y_cw)),
                                 jnp.max(jnp.abs(y_ccw)))
        amax_buf[me] = jnp.full((8, 128), local_amax, jnp.float32)
        sends = []
        for o in range(1, N_DEV):
            tgt = lax.rem(me + o, N_DEV)
            s = pltpu.make_async_remote_copy(
                src_ref=amax_buf.at[me], dst_ref=amax_buf.at[me],
                send_sem=amax_send_sems.at[o], recv_sem=amax_recv_sems.at[me],
                device_id=(tgt,), device_id_type=pl.DeviceIdType.MESH)
            s.start()
            sends.append(s)
        for o in range(1, N_DEV):
            src_dev = lax.rem(me + N_DEV - o, N_DEV)
            r = pltpu.make_async_remote_copy(
                src_ref=amax_buf.at[me], dst_ref=amax_buf.at[src_dev],
                send_sem=amax_send_sems.at[0],
                recv_sem=amax_recv_sems.at[src_dev],
                device_id=(left,), device_id_type=pl.DeviceIdType.MESH)
            r.wait_recv()

        g_amax = jnp.max(amax_buf[...])
        scale = g_amax / 127.0
        q = jnp.clip(jnp.round(out_ref[...] / scale), -127.0, 127.0)
        out_ref[...] = q * scale

        for s in sends:
            s.wait_send()

        for nbr in (left, right):
            pl.semaphore_signal(exit_sem, inc=1, device_id=(nbr,),
                                device_id_type=pl.DeviceIdType.MESH)
        pl.semaphore_wait(exit_sem, 2)

    return pl.pallas_call(
        body,
        out_shape=jax.ShapeDtypeStruct((m_per, n), jnp.float32),
        in_specs=[pl.BlockSpec(memory_space=pltpu.VMEM),
                  pl.BlockSpec(memory_space=pltpu.VMEM)],
        out_specs=pl.BlockSpec(memory_space=pltpu.VMEM),
        scratch_shapes=[
            pltpu.VMEM((m_per, n_half), jnp.bfloat16),
            pltpu.VMEM((m_per, n_half), jnp.bfloat16),
            pltpu.VMEM((m_per, n_half), jnp.bfloat16),
            pltpu.VMEM((m_per, n_half), jnp.bfloat16),
            pltpu.SemaphoreType.DMA((2,)),
            pltpu.SemaphoreType.DMA((2,)),
            pltpu.SemaphoreType.REGULAR,
            pltpu.SemaphoreType.REGULAR,
            pltpu.VMEM((N_DEV, 8, 128), jnp.float32),
            pltpu.SemaphoreType.DMA((N_DEV,)),
            pltpu.SemaphoreType.DMA((N_DEV,)),
            pltpu.SemaphoreType.REGULAR,
        ],
        compiler_params=pltpu.CompilerParams(
            collective_id=0,
            vmem_limit_bytes=100 * 1024 * 1024,
        ),
    )(x, w_mat)


# device time: 368939 ns/iter; 1.2566x vs baseline; 1.2562x over previous
import jax
import jax.numpy as jnp
from jax import lax
from jax.experimental import pallas as pl
from jax.experimental.pallas import tpu as pltpu

N_DEV = 16
N_SUB = 2


def kernel(x, w_mat):
    m, k_per = x.shape
    _, n = w_mat.shape
    m_per = m // N_DEV
    n_half = n // 2
    n_sub = n_half // N_SUB

    def body(x_ref, w_ref, out_ref, *scratch):
        send_bufs = [[scratch[2 * d + j] for j in range(N_SUB)]
                     for d in range(2)]
        recv_bufs = [[scratch[4 + 2 * d + j] for j in range(N_SUB)]
                     for d in range(2)]
        (send_sems, recv_sems, credit_sems,
         amax_buf, amax_send_sems, amax_recv_sems, exit_sem) = scratch[8:]

        me = lax.axis_index("i")
        left = lax.rem(me + N_DEV - 1, N_DEV)
        right = lax.rem(me + 1, N_DEV)
        tgt = [right, left]
        peer = [left, right]

        barrier = pltpu.get_barrier_semaphore()
        for nbr in (left, right):
            pl.semaphore_signal(barrier, inc=1, device_id=(nbr,),
                                device_id_type=pl.DeviceIdType.MESH)
        pl.semaphore_wait(barrier, 2)

        def chunk(d, h):
            return lax.rem(me + 2 * N_DEV - 1 - h, N_DEV) if d == 0 \
                else lax.rem(me + 1 + h, N_DEV)

        def partial(c, d, j):
            xa = x_ref[pl.ds(c * m_per, m_per), :]
            lo = d * n_half + j * n_sub
            return jnp.dot(xa, w_ref[:, lo:lo + n_sub],
                           preferred_element_type=jnp.float32)

        def make_rdma(d, j):
            return pltpu.make_async_remote_copy(
                src_ref=send_bufs[d][j], dst_ref=recv_bufs[d][j],
                send_sem=send_sems.at[d, j], recv_sem=recv_sems.at[d, j],
                device_id=(tgt[d],), device_id_type=pl.DeviceIdType.MESH)

        order = [(0, 0), (1, 0), (0, 1), (1, 1)]
        prev = {}
        for d, j in order:
            send_bufs[d][j][...] = partial(chunk(d, 0), d, j
                                           ).astype(jnp.bfloat16)
            r = make_rdma(d, j)
            r.start()
            prev[d, j] = r

        for h in range(1, N_DEV - 1):
            for d, j in order:
                prev[d, j].wait()
                acc = partial(chunk(d, h), d, j) \
                    + recv_bufs[d][j][...].astype(jnp.float32)
                send_bufs[d][j][...] = acc.astype(jnp.bfloat16)
                pl.semaphore_signal(credit_sems.at[d, j], inc=1,
                                    device_id=(peer[d],),
                                    device_id_type=pl.DeviceIdType.MESH)
                pl.semaphore_wait(credit_sems.at[d, j], 1)
                r = make_rdma(d, j)
                r.start()
                prev[d, j] = r

        local_amax = jnp.float32(0.0)
        for d, j in order:
            prev[d, j].wait()
            y = partial(me, d, j) + recv_bufs[d][j][...].astype(jnp.float32)
            lo = d * n_half + j * n_sub
            out_ref[:, lo:lo + n_sub] = y
            local_amax = jnp.maximum(local_amax, jnp.max(jnp.abs(y)))

        amax_buf[me] = jnp.full((8, 128), local_amax, jnp.float32)
        sends = []
        for o in range(1, N_DEV):
            t = lax.rem(me + o, N_DEV)
            s = pltpu.make_async_remote_copy(
                src_ref=amax_buf.at[me], dst_ref=amax_buf.at[me],
                send_sem=amax_send_sems.at[o], recv_sem=amax_recv_sems.at[me],
                device_id=(t,), device_id_type=pl.DeviceIdType.MESH)
            s.start()
            sends.append(s)
        for o in range(1, N_DEV):
            src_dev = lax.rem(me + N_DEV - o, N_DEV)
            r = pltpu.make_async_remote_copy(
                src_ref=amax_buf.at[me], dst_ref=amax_buf.at[src_dev],
                send_sem=amax_send_sems.at[0],
                recv_sem=amax_recv_sems.at[src_dev],
                device_id=(left,), device_id_type=pl.DeviceIdType.MESH)
            r.wait_recv()

        g_amax = jnp.max(amax_buf[...])
        scale = g_amax / 127.0
        q = jnp.clip(jnp.round(out_ref[...] / scale), -127.0, 127.0)
        out_ref[...] = q * scale

        for s in sends:
            s.wait_send()

        for nbr in (left, right):
            pl.semaphore_signal(exit_sem, inc=1, device_id=(nbr,),
                                device_id_type=pl.DeviceIdType.MESH)
        pl.semaphore_wait(exit_sem, 2)

    comm_buf = pltpu.VMEM((m_per, n_sub), jnp.bfloat16)
    return pl.pallas_call(
        body,
        out_shape=jax.ShapeDtypeStruct((m_per, n), jnp.float32),
        in_specs=[pl.BlockSpec(memory_space=pltpu.VMEM),
                  pl.BlockSpec(memory_space=pltpu.VMEM)],
        out_specs=pl.BlockSpec(memory_space=pltpu.VMEM),
        scratch_shapes=[
            comm_buf, comm_buf, comm_buf, comm_buf,
            comm_buf, comm_buf, comm_buf, comm_buf,
            pltpu.SemaphoreType.DMA((2, N_SUB)),
            pltpu.SemaphoreType.DMA((2, N_SUB)),
            pltpu.SemaphoreType.REGULAR((2, N_SUB)),
            pltpu.VMEM((N_DEV, 8, 128), jnp.float32),
            pltpu.SemaphoreType.DMA((N_DEV,)),
            pltpu.SemaphoreType.DMA((N_DEV,)),
            pltpu.SemaphoreType.REGULAR,
        ],
        compiler_params=pltpu.CompilerParams(
            collective_id=0,
            vmem_limit_bytes=100 * 1024 * 1024,
        ),
    )(x, w_mat)


# device time: 368814 ns/iter; 1.2571x vs baseline; 1.0003x over previous
import jax
import jax.numpy as jnp
from jax import lax
from jax.experimental import pallas as pl
from jax.experimental.pallas import tpu as pltpu

N_DEV = 16
N_SUB = 4


def kernel(x, w_mat):
    m, k_per = x.shape
    _, n = w_mat.shape
    m_per = m // N_DEV
    n_half = n // 2
    n_sub = n_half // N_SUB

    def body(x_ref, w_ref, out_ref, *scratch):
        send_bufs = [[scratch[N_SUB * d + j] for j in range(N_SUB)]
                     for d in range(2)]
        recv_bufs = [[scratch[2 * N_SUB + N_SUB * d + j] for j in range(N_SUB)]
                     for d in range(2)]
        (send_sems, recv_sems, credit_sems,
         amax_buf, amax_send_sems, amax_recv_sems, exit_sem) = scratch[4 * N_SUB:]

        me = lax.axis_index("i")
        left = lax.rem(me + N_DEV - 1, N_DEV)
        right = lax.rem(me + 1, N_DEV)
        tgt = [right, left]
        peer = [left, right]

        barrier = pltpu.get_barrier_semaphore()
        for nbr in (left, right):
            pl.semaphore_signal(barrier, inc=1, device_id=(nbr,),
                                device_id_type=pl.DeviceIdType.MESH)
        pl.semaphore_wait(barrier, 2)

        def chunk(d, h):
            return lax.rem(me + 2 * N_DEV - 1 - h, N_DEV) if d == 0 \
                else lax.rem(me + 1 + h, N_DEV)

        def partial(c, d, j):
            xa = x_ref[pl.ds(c * m_per, m_per), :]
            lo = d * n_half + j * n_sub
            return jnp.dot(xa, w_ref[:, lo:lo + n_sub],
                           preferred_element_type=jnp.float32)

        def make_rdma(d, j):
            return pltpu.make_async_remote_copy(
                src_ref=send_bufs[d][j], dst_ref=recv_bufs[d][j],
                send_sem=send_sems.at[d, j], recv_sem=recv_sems.at[d, j],
                device_id=(tgt[d],), device_id_type=pl.DeviceIdType.MESH)

        order = [(d, j) for j in range(N_SUB) for d in range(2)]
        prev = {}
        for d, j in order:
            send_bufs[d][j][...] = partial(chunk(d, 0), d, j
                                           ).astype(jnp.bfloat16)
            r = make_rdma(d, j)
            r.start()
            prev[d, j] = r

        for h in range(1, N_DEV - 1):
            for d, j in order:
                prev[d, j].wait()
                acc = partial(chunk(d, h), d, j) \
                    + recv_bufs[d][j][...].astype(jnp.float32)
                send_bufs[d][j][...] = acc.astype(jnp.bfloat16)
                pl.semaphore_signal(credit_sems.at[d, j], inc=1,
                                    device_id=(peer[d],),
                                    device_id_type=pl.DeviceIdType.MESH)
                pl.semaphore_wait(credit_sems.at[d, j], 1)
                r = make_rdma(d, j)
                r.start()
                prev[d, j] = r

        local_amax = jnp.float32(0.0)
        for d, j in order:
            prev[d, j].wait()
            y = partial(me, d, j) + recv_bufs[d][j][...].astype(jnp.float32)
            lo = d * n_half + j * n_sub
            out_ref[:, lo:lo + n_sub] = y
            local_amax = jnp.maximum(local_amax, jnp.max(jnp.abs(y)))

        amax_buf[me] = jnp.full((8, 128), local_amax, jnp.float32)
        sends = []
        for o in range(1, N_DEV):
            t = lax.rem(me + o, N_DEV)
            s = pltpu.make_async_remote_copy(
                src_ref=amax_buf.at[me], dst_ref=amax_buf.at[me],
                send_sem=amax_send_sems.at[o], recv_sem=amax_recv_sems.at[me],
                device_id=(t,), device_id_type=pl.DeviceIdType.MESH)
            s.start()
            sends.append(s)
        for o in range(1, N_DEV):
            src_dev = lax.rem(me + N_DEV - o, N_DEV)
            r = pltpu.make_async_remote_copy(
                src_ref=amax_buf.at[me], dst_ref=amax_buf.at[src_dev],
                send_sem=amax_send_sems.at[0],
                recv_sem=amax_recv_sems.at[src_dev],
                device_id=(left,), device_id_type=pl.DeviceIdType.MESH)
            r.wait_recv()

        g_amax = jnp.max(amax_buf[...])
        scale = g_amax / 127.0
        q = jnp.clip(jnp.round(out_ref[...] / scale), -127.0, 127.0)
        out_ref[...] = q * scale

        for s in sends:
            s.wait_send()

        for nbr in (left, right):
            pl.semaphore_signal(exit_sem, inc=1, device_id=(nbr,),
                                device_id_type=pl.DeviceIdType.MESH)
        pl.semaphore_wait(exit_sem, 2)

    comm_buf = pltpu.VMEM((m_per, n_sub), jnp.bfloat16)
    return pl.pallas_call(
        body,
        out_shape=jax.ShapeDtypeStruct((m_per, n), jnp.float32),
        in_specs=[pl.BlockSpec(memory_space=pltpu.VMEM),
                  pl.BlockSpec(memory_space=pltpu.VMEM)],
        out_specs=pl.BlockSpec(memory_space=pltpu.VMEM),
        scratch_shapes=[
            *([comm_buf] * (2 * N_SUB)),
            *([comm_buf] * (2 * N_SUB)),
            pltpu.SemaphoreType.DMA((2, N_SUB)),
            pltpu.SemaphoreType.DMA((2, N_SUB)),
            pltpu.SemaphoreType.REGULAR((2, N_SUB)),
            pltpu.VMEM((N_DEV, 8, 128), jnp.float32),
            pltpu.SemaphoreType.DMA((N_DEV,)),
            pltpu.SemaphoreType.DMA((N_DEV,)),
            pltpu.SemaphoreType.REGULAR,
        ],
        compiler_params=pltpu.CompilerParams(
            collective_id=0,
            vmem_limit_bytes=100 * 1024 * 1024,
        ),
    )(x, w_mat)
